# baseline (speedup 1.0000x reference)
import functools

import jax
import jax.numpy as jnp
from jax import lax
from jax.experimental import pallas as pl
from jax.experimental.pallas import tpu as pltpu
from jax.experimental.pallas import tpu_sc as plsc

T = 2048
D = 1024
E = 8
K = 1024
K2 = 2 * K
TB = 128
NJOBS = T // TB + E
P = T + E * TB
BALANCE_COEF = 0.01
LN_EPS = 1e-5

SC_CORES = 2
SC_SUBCORES = 16
SC_WORKERS = SC_CORES * SC_SUBCORES
CHUNK = T // SC_WORKERS


def _gating_body(x_ref, w1_ref, b1_ref, w2_ref, pos_ref, tbl_ref, loss_ref):
    xt = x_ref[...]
    g1 = lax.dot_general(xt, w1_ref[...], (((1,), (1,)), ((), ())),
                         preferred_element_type=jnp.float32) + b1_ref[...]
    ga = g1[:, :E] * jax.nn.sigmoid(g1[:, E:])
    logits = lax.dot_general(ga, w2_ref[...], (((1,), (1,)), ((), ())),
                             preferred_element_type=jnp.float32)

    ie = lax.broadcasted_iota(jnp.int32, (T, E), 1)
    mx = jnp.max(logits, axis=1, keepdims=True)
    choice = jnp.min(jnp.where(logits == mx, ie, E), axis=1, keepdims=True)
    onehot = (ie == choice).astype(jnp.float32)

    counts = jnp.sum(onehot, axis=0, keepdims=True)
    probs = counts * (1.0 / T)
    plogp = probs * jnp.log(probs + 1e-10)
    loss_ref[...] = jnp.sum(plogp, axis=1, keepdims=True) * (-BALANCE_COEF)

    C = 16
    CH = T // C
    r = lax.broadcasted_iota(jnp.int32, (CH, CH), 0)
    c = lax.broadcasted_iota(jnp.int32, (CH, CH), 1)
    stril = (c < r).astype(jnp.float32)
    parts = [
        lax.dot_general(stril, onehot[i * CH:(i + 1) * CH, :],
                        (((1,), (0,)), ((), ())),
                        preferred_element_type=jnp.float32)
        for i in range(C)
    ]
    within = jnp.concatenate(parts, axis=0)
    rA = lax.broadcasted_iota(jnp.int32, (C, T), 0)
    cA = lax.broadcasted_iota(jnp.int32, (C, T), 1)
    A = (cA // CH == rA).astype(jnp.float32)
    csum = lax.dot_general(A, onehot, (((1,), (0,)), ((), ())),
                           preferred_element_type=jnp.float32)
    r16 = lax.broadcasted_iota(jnp.int32, (C, C), 0)
    c16 = lax.broadcasted_iota(jnp.int32, (C, C), 1)
    stril16 = (c16 < r16).astype(jnp.float32)
    coff = lax.dot_general(stril16, csum, (((1,), (0,)), ((), ())),
                           preferred_element_type=jnp.float32)
    rT = lax.broadcasted_iota(jnp.int32, (T, C), 0)
    cT = lax.broadcasted_iota(jnp.int32, (T, C), 1)
    AT = (rT // CH == cT).astype(jnp.float32)
    coff_hi = jnp.floor(coff * (1.0 / CH))
    coff_lo = coff - coff_hi * CH
    rank = within + (
        lax.dot_general(AT, coff_hi, (((1,), (0,)), ((), ())),
                        preferred_element_type=jnp.float32) * CH
        + lax.dot_general(AT, coff_lo, (((1,), (0,)), ((), ())),
                          preferred_element_type=jnp.float32))

    re8 = lax.broadcasted_iota(jnp.int32, (E, E), 0)
    ce8 = lax.broadcasted_iota(jnp.int32, (E, E), 1)
    striu = (re8 < ce8).astype(jnp.float32)
    striu_inc = (re8 <= ce8).astype(jnp.float32)
    jobs = jnp.floor((counts + (TB - 1)) * (1.0 / TB))
    aoff = lax.dot_general(jobs, striu, (((1,), (0,)), ((), ())),
                           preferred_element_type=jnp.float32) * TB
    posf = jnp.sum(onehot * (rank + aoff), axis=1, keepdims=True)
    pos_ref[...] = posf.astype(jnp.int32)

    jinc = lax.dot_general(jobs, striu_inc, (((1,), (0,)), ((), ())),
                           preferred_element_type=jnp.float32)
    jexc = lax.dot_general(jobs, striu, (((1,), (0,)), ((), ())),
                           preferred_element_type=jnp.float32)
    total = jnp.sum(jobs)
    jv = lax.broadcasted_iota(jnp.int32, (NJOBS, E), 0).astype(jnp.float32)
    jcl = jnp.minimum(jv, total - 1.0)
    jexp = jnp.sum((jcl >= jinc).astype(jnp.float32), axis=1, keepdims=True)
    ienj = lax.broadcasted_iota(jnp.int32, (NJOBS, E), 1).astype(jnp.float32)
    ohj = (ienj == jexp).astype(jnp.float32)
    jc_j = jnp.sum(ohj * jexc, axis=1, keepdims=True)
    aoff_j = jnp.sum(ohj * aoff, axis=1, keepdims=True)
    jtile = aoff_j * (1.0 / TB) + (jcl[:, :1] - jc_j)
    validf = (jv[:, :1] < total).astype(jnp.float32)
    tbl_ref[...] = jnp.concatenate([jexp, jtile, validf], axis=1).astype(jnp.int32)


def _gating(flat_x, gate_w1, gate_b1, gate_w2):
    return pl.pallas_call(
        _gating_body,
        out_shape=[
            jax.ShapeDtypeStruct((T, 1), jnp.int32),
            jax.ShapeDtypeStruct((NJOBS, 3), jnp.int32),
            jax.ShapeDtypeStruct((1, 1), jnp.float32),
        ],
    )(flat_x, gate_w1, gate_b1.reshape(1, 2 * E), gate_w2)


def _expert_body(s_ref, x_ref, w_ref, b_ref, g_ref, bb_ref, pw_ref, pb_ref, o_ref):
    j = pl.program_id(0)

    @pl.when(s_ref[j, 2] == 1)
    def _():
        xt = x_ref[...].astype(jnp.bfloat16)
        w = w_ref[0]
        h = lax.dot_general(xt, w, (((1,), (1,)), ((), ())),
                            preferred_element_type=jnp.float32) + b_ref[0]
        glu = h[:, :K] * jax.nn.sigmoid(h[:, K:])
        m = jnp.mean(glu, axis=1, keepdims=True)
        v = jnp.mean((glu - m) * (glu - m), axis=1, keepdims=True)
        normed = (glu - m) * lax.rsqrt(v + LN_EPS)
        oe = (normed * g_ref[0] + bb_ref[0]).astype(jnp.bfloat16)
        y = lax.dot_general(oe, pw_ref[...], (((1,), (1,)), ((), ())),
                            preferred_element_type=jnp.float32)
        o_ref[...] = y + pb_ref[...]


def _expert_call(tbl, x_sorted, ew1, eb1, ln_g, ln_b, proj_w, proj_b2):
    grid_spec = pltpu.PrefetchScalarGridSpec(
        num_scalar_prefetch=1,
        grid=(NJOBS,),
        in_specs=[
            pl.BlockSpec((TB, D), lambda j, s: (s[j, 1], 0)),
            pl.BlockSpec((1, K2, D), lambda j, s: (s[j, 0], 0, 0)),
            pl.BlockSpec((1, 1, K2), lambda j, s: (s[j, 0], 0, 0)),
            pl.BlockSpec((1, 1, K), lambda j, s: (s[j, 0], 0, 0)),
            pl.BlockSpec((1, 1, K), lambda j, s: (s[j, 0], 0, 0)),
            pl.BlockSpec((D, K), lambda j, s: (0, 0)),
            pl.BlockSpec((1, D), lambda j, s: (0, 0)),
        ],
        out_specs=pl.BlockSpec((TB, D), lambda j, s: (s[j, 1], 0)),
    )
    return pl.pallas_call(
        _expert_body,
        grid_spec=grid_spec,
        out_shape=jax.ShapeDtypeStruct((P, D), jnp.float32),
    )(tbl, x_sorted, ew1.astype(jnp.bfloat16), eb1.reshape(E, 1, K2),
      ln_g.reshape(E, 1, K), ln_b.reshape(E, 1, K),
      proj_w.astype(jnp.bfloat16), proj_b2)


def _sc_mesh():
    return plsc.VectorSubcoreMesh(core_axis_name="c", subcore_axis_name="s",
                                  num_cores=SC_CORES, num_subcores=SC_SUBCORES)


def _sc_dispatch(flat_x, pos):
    @functools.partial(
        pl.kernel,
        out_type=jax.ShapeDtypeStruct((P, D), jnp.float32),
        mesh=_sc_mesh(),
        scratch_types=[
            pltpu.VMEM((CHUNK,), jnp.int32),
            pltpu.VMEM((CHUNK, D), jnp.float32),
            pltpu.SemaphoreType.DMA,
        ],
    )
    def k(x_hbm, pos_hbm, out_hbm, idx_v, rows_v, sem):
        wid = lax.axis_index("s") * SC_CORES + lax.axis_index("c")
        base = wid * CHUNK
        pltpu.sync_copy(pos_hbm.at[pl.ds(base, CHUNK)], idx_v)
        pltpu.sync_copy(x_hbm.at[pl.ds(base, CHUNK)], rows_v)
        pltpu.async_copy(rows_v, out_hbm.at[idx_v], sem).wait()

    return k(flat_x, pos)


def _sc_unpermute(y_sorted, pos):
    @functools.partial(
        pl.kernel,
        out_type=jax.ShapeDtypeStruct((T, D), jnp.float32),
        mesh=_sc_mesh(),
        scratch_types=[
            pltpu.VMEM((CHUNK,), jnp.int32),
            pltpu.VMEM((CHUNK, D), jnp.float32),
            pltpu.SemaphoreType.DMA,
        ],
    )
    def k(y_hbm, pos_hbm, out_hbm, idx_v, rows_v, sem):
        wid = lax.axis_index("s") * SC_CORES + lax.axis_index("c")
        base = wid * CHUNK
        pltpu.sync_copy(pos_hbm.at[pl.ds(base, CHUNK)], idx_v)
        pltpu.async_copy(y_hbm.at[idx_v], rows_v, sem).wait()
        pltpu.sync_copy(rows_v, out_hbm.at[pl.ds(base, CHUNK)])

    return k(y_sorted, pos)


def kernel(x, gate_w1, gate_b1, gate_w2, ew1, eb1, ln_g, ln_b, proj_w, proj_b):
    bs, seq, d = x.shape
    flat_x = x.reshape(bs * seq, d)
    pos2, tbl, loss = _gating(flat_x, gate_w1, gate_b1, gate_w2)
    pos = pos2.reshape(T)
    x_sorted = _sc_dispatch(flat_x, pos)
    y_sorted = _expert_call(tbl, x_sorted, ew1, eb1, ln_g, ln_b,
                            proj_w, proj_b.reshape(1, D))
    y = _sc_unpermute(y_sorted, pos)
    return y.reshape(bs, seq, d), loss[0, 0]


# speedup vs baseline: 1.5085x; 1.5085x over previous
import functools

import jax
import jax.numpy as jnp
from jax import lax
from jax.experimental import pallas as pl
from jax.experimental.pallas import tpu as pltpu
from jax.experimental.pallas import tpu_sc as plsc

T = 2048
D = 1024
E = 8
K = 1024
K2 = 2 * K
TB = 256
NJOBS = T // TB + E
P = T + E * TB
BALANCE_COEF = 0.01
LN_EPS = 1e-5

SC_CORES = 2
SC_SUBCORES = 16
SC_WORKERS = SC_CORES * SC_SUBCORES
CHUNK = T // SC_WORKERS


def _gating_body(x_ref, w1_ref, b1_ref, w2_ref, pos_ref, tbl_ref, loss_ref):
    xt = x_ref[...]
    g1 = lax.dot_general(xt, w1_ref[...], (((1,), (1,)), ((), ())),
                         preferred_element_type=jnp.float32) + b1_ref[...]
    ga = g1[:, :E] * jax.nn.sigmoid(g1[:, E:])
    logits = lax.dot_general(ga, w2_ref[...], (((1,), (1,)), ((), ())),
                             preferred_element_type=jnp.float32)

    ie = lax.broadcasted_iota(jnp.int32, (T, E), 1)
    mx = jnp.max(logits, axis=1, keepdims=True)
    choice = jnp.min(jnp.where(logits == mx, ie, E), axis=1, keepdims=True)
    onehot = (ie == choice).astype(jnp.float32)

    counts = jnp.sum(onehot, axis=0, keepdims=True)
    probs = counts * (1.0 / T)
    plogp = probs * jnp.log(probs + 1e-10)
    loss_ref[...] = jnp.sum(plogp, axis=1, keepdims=True) * (-BALANCE_COEF)

    C = 16
    CH = T // C
    r = lax.broadcasted_iota(jnp.int32, (CH, CH), 0)
    c = lax.broadcasted_iota(jnp.int32, (CH, CH), 1)
    stril = (c < r).astype(jnp.float32)
    parts = [
        lax.dot_general(stril, onehot[i * CH:(i + 1) * CH, :],
                        (((1,), (0,)), ((), ())),
                        preferred_element_type=jnp.float32)
        for i in range(C)
    ]
    within = jnp.concatenate(parts, axis=0)
    rA = lax.broadcasted_iota(jnp.int32, (C, T), 0)
    cA = lax.broadcasted_iota(jnp.int32, (C, T), 1)
    A = (cA // CH == rA).astype(jnp.float32)
    csum = lax.dot_general(A, onehot, (((1,), (0,)), ((), ())),
                           preferred_element_type=jnp.float32)
    r16 = lax.broadcasted_iota(jnp.int32, (C, C), 0)
    c16 = lax.broadcasted_iota(jnp.int32, (C, C), 1)
    stril16 = (c16 < r16).astype(jnp.float32)
    coff = lax.dot_general(stril16, csum, (((1,), (0,)), ((), ())),
                           preferred_element_type=jnp.float32)
    rT = lax.broadcasted_iota(jnp.int32, (T, C), 0)
    cT = lax.broadcasted_iota(jnp.int32, (T, C), 1)
    AT = (rT // CH == cT).astype(jnp.float32)
    coff_hi = jnp.floor(coff * (1.0 / CH))
    coff_lo = coff - coff_hi * CH
    rank = within + (
        lax.dot_general(AT, coff_hi, (((1,), (0,)), ((), ())),
                        preferred_element_type=jnp.float32) * CH
        + lax.dot_general(AT, coff_lo, (((1,), (0,)), ((), ())),
                          preferred_element_type=jnp.float32))

    re8 = lax.broadcasted_iota(jnp.int32, (E, E), 0)
    ce8 = lax.broadcasted_iota(jnp.int32, (E, E), 1)
    striu = (re8 < ce8).astype(jnp.float32)
    striu_inc = (re8 <= ce8).astype(jnp.float32)
    jobs = jnp.floor((counts + (TB - 1)) * (1.0 / TB))
    aoff = lax.dot_general(jobs, striu, (((1,), (0,)), ((), ())),
                           preferred_element_type=jnp.float32) * TB
    posf = jnp.sum(onehot * (rank + aoff), axis=1, keepdims=True)
    pos_ref[...] = posf.astype(jnp.int32)

    jinc = lax.dot_general(jobs, striu_inc, (((1,), (0,)), ((), ())),
                           preferred_element_type=jnp.float32)
    jexc = lax.dot_general(jobs, striu, (((1,), (0,)), ((), ())),
                           preferred_element_type=jnp.float32)
    total = jnp.sum(jobs)
    jv = lax.broadcasted_iota(jnp.int32, (NJOBS, E), 0).astype(jnp.float32)
    jcl = jnp.minimum(jv, total - 1.0)
    jexp = jnp.sum((jcl >= jinc).astype(jnp.float32), axis=1, keepdims=True)
    ienj = lax.broadcasted_iota(jnp.int32, (NJOBS, E), 1).astype(jnp.float32)
    ohj = (ienj == jexp).astype(jnp.float32)
    jc_j = jnp.sum(ohj * jexc, axis=1, keepdims=True)
    aoff_j = jnp.sum(ohj * aoff, axis=1, keepdims=True)
    jtile = aoff_j * (1.0 / TB) + (jcl[:, :1] - jc_j)
    validf = (jv[:, :1] < total).astype(jnp.float32)
    tbl_ref[...] = jnp.concatenate([jexp, jtile, validf], axis=1).astype(jnp.int32)


def _gating(flat_x, gate_w1, gate_b1, gate_w2):
    return pl.pallas_call(
        _gating_body,
        out_shape=[
            jax.ShapeDtypeStruct((T, 1), jnp.int32),
            jax.ShapeDtypeStruct((NJOBS, 3), jnp.int32),
            jax.ShapeDtypeStruct((1, 1), jnp.float32),
        ],
    )(flat_x, gate_w1, gate_b1.reshape(1, 2 * E), gate_w2)


def _expert_body(s_ref, x_ref, w_ref, b_ref, g_ref, bb_ref, pw_ref, pb_ref, o_ref):
    j = pl.program_id(0)

    @pl.when(s_ref[j, 2] == 1)
    def _():
        xt = x_ref[...]
        w = w_ref[0]
        h = lax.dot_general(xt, w, (((1,), (1,)), ((), ())),
                            preferred_element_type=jnp.float32) + b_ref[0]
        glu = h[:, :K] * jax.nn.sigmoid(h[:, K:])
        m = jnp.mean(glu, axis=1, keepdims=True)
        v = jnp.mean((glu - m) * (glu - m), axis=1, keepdims=True)
        normed = (glu - m) * lax.rsqrt(v + LN_EPS)
        oe = normed * g_ref[0] + bb_ref[0]
        y = lax.dot_general(oe, pw_ref[...], (((1,), (1,)), ((), ())),
                            preferred_element_type=jnp.float32)
        o_ref[...] = y + pb_ref[...]


def _expert_call(tbl, x_sorted, ew1, eb1, ln_g, ln_b, proj_w, proj_b2):
    grid_spec = pltpu.PrefetchScalarGridSpec(
        num_scalar_prefetch=1,
        grid=(NJOBS,),
        in_specs=[
            pl.BlockSpec((TB, D), lambda j, s: (s[j, 1], 0)),
            pl.BlockSpec((1, K2, D), lambda j, s: (s[j, 0], 0, 0)),
            pl.BlockSpec((1, 1, K2), lambda j, s: (s[j, 0], 0, 0)),
            pl.BlockSpec((1, 1, K), lambda j, s: (s[j, 0], 0, 0)),
            pl.BlockSpec((1, 1, K), lambda j, s: (s[j, 0], 0, 0)),
            pl.BlockSpec((D, K), lambda j, s: (0, 0)),
            pl.BlockSpec((1, D), lambda j, s: (0, 0)),
        ],
        out_specs=pl.BlockSpec((TB, D), lambda j, s: (s[j, 1], 0)),
    )
    return pl.pallas_call(
        _expert_body,
        grid_spec=grid_spec,
        out_shape=jax.ShapeDtypeStruct((P, D), jnp.float32),
    )(tbl, x_sorted, ew1, eb1.reshape(E, 1, K2), ln_g.reshape(E, 1, K),
      ln_b.reshape(E, 1, K), proj_w, proj_b2)


def _sc_mesh():
    return plsc.VectorSubcoreMesh(core_axis_name="c", subcore_axis_name="s",
                                  num_cores=SC_CORES, num_subcores=SC_SUBCORES)


def _sc_dispatch(flat_x, pos):
    @functools.partial(
        pl.kernel,
        out_type=jax.ShapeDtypeStruct((P, D), jnp.float32),
        mesh=_sc_mesh(),
        scratch_types=[
            pltpu.VMEM((CHUNK,), jnp.int32),
            pltpu.VMEM((CHUNK, D), jnp.float32),
            pltpu.SemaphoreType.DMA,
        ],
    )
    def k(x_hbm, pos_hbm, out_hbm, idx_v, rows_v, sem):
        wid = lax.axis_index("s") * SC_CORES + lax.axis_index("c")
        base = wid * CHUNK
        pltpu.sync_copy(pos_hbm.at[pl.ds(base, CHUNK)], idx_v)
        pltpu.sync_copy(x_hbm.at[pl.ds(base, CHUNK)], rows_v)
        pltpu.async_copy(rows_v, out_hbm.at[idx_v], sem).wait()

    return k(flat_x, pos)


def _sc_unpermute(y_sorted, pos):
    @functools.partial(
        pl.kernel,
        out_type=jax.ShapeDtypeStruct((T, D), jnp.float32),
        mesh=_sc_mesh(),
        scratch_types=[
            pltpu.VMEM((CHUNK,), jnp.int32),
            pltpu.VMEM((CHUNK, D), jnp.float32),
            pltpu.SemaphoreType.DMA,
        ],
    )
    def k(y_hbm, pos_hbm, out_hbm, idx_v, rows_v, sem):
        wid = lax.axis_index("s") * SC_CORES + lax.axis_index("c")
        base = wid * CHUNK
        pltpu.sync_copy(pos_hbm.at[pl.ds(base, CHUNK)], idx_v)
        pltpu.async_copy(y_hbm.at[idx_v], rows_v, sem).wait()
        pltpu.sync_copy(rows_v, out_hbm.at[pl.ds(base, CHUNK)])

    return k(y_sorted, pos)


def kernel(x, gate_w1, gate_b1, gate_w2, ew1, eb1, ln_g, ln_b, proj_w, proj_b):
    bs, seq, d = x.shape
    flat_x = x.reshape(bs * seq, d)
    pos2, tbl, loss = _gating(flat_x, gate_w1, gate_b1, gate_w2)
    pos = pos2.reshape(T)
    x_sorted = _sc_dispatch(flat_x, pos)
    y_sorted = _expert_call(tbl, x_sorted, ew1, eb1, ln_g, ln_b,
                            proj_w, proj_b.reshape(1, D))
    y = _sc_unpermute(y_sorted, pos)
    return y.reshape(bs, seq, d), loss[0, 0]


# speedup vs baseline: 1.6160x; 1.0713x over previous
import functools

import jax
import jax.numpy as jnp
from jax import lax
from jax.experimental import pallas as pl
from jax.experimental.pallas import tpu as pltpu
from jax.experimental.pallas import tpu_sc as plsc

T = 2048
D = 1024
E = 8
K = 1024
K2 = 2 * K
TB = 256
NJOBS = T // TB + E
P = T + E * TB
BALANCE_COEF = 0.01
LN_EPS = 1e-5

SC_CORES = 2
SC_SUBCORES = 16
SC_WORKERS = SC_CORES * SC_SUBCORES
CHUNK = T // SC_WORKERS


def _gating_body(x_ref, w1_ref, b1_ref, w2_ref, pos_ref, tbl_ref, loss_ref):
    xt = x_ref[...]
    g1 = lax.dot_general(xt, w1_ref[...], (((1,), (1,)), ((), ())),
                         preferred_element_type=jnp.float32) + b1_ref[...]
    ga = g1[:, :E] * jax.nn.sigmoid(g1[:, E:])
    lt = lax.dot_general(w2_ref[...], ga, (((1,), (1,)), ((), ())),
                         preferred_element_type=jnp.float32)

    ie = lax.broadcasted_iota(jnp.int32, (E, T), 0)
    mx = jnp.max(lt, axis=0, keepdims=True)
    choice = jnp.min(jnp.where(lt == mx, ie, E), axis=0, keepdims=True)
    onehot = (ie == choice).astype(jnp.float32)

    counts = jnp.sum(onehot, axis=1, keepdims=True)
    probs = counts * (1.0 / T)
    plogp = probs * jnp.log(probs + 1e-10)
    loss_ref[...] = jnp.sum(plogp, axis=0, keepdims=True) * (-BALANCE_COEF)

    C = 16
    CH = T // C
    r = lax.broadcasted_iota(jnp.int32, (CH, CH), 0)
    c = lax.broadcasted_iota(jnp.int32, (CH, CH), 1)
    striu = (r < c).astype(jnp.float32)
    parts = [
        lax.dot_general(onehot[:, i * CH:(i + 1) * CH], striu,
                        (((1,), (0,)), ((), ())),
                        preferred_element_type=jnp.float32)
        for i in range(C)
    ]
    within = jnp.concatenate(parts, axis=1)
    rA = lax.broadcasted_iota(jnp.int32, (T, C), 0)
    cA = lax.broadcasted_iota(jnp.int32, (T, C), 1)
    AT = (rA // CH == cA).astype(jnp.float32)
    csum = lax.dot_general(onehot, AT, (((1,), (0,)), ((), ())),
                           preferred_element_type=jnp.float32)
    r16 = lax.broadcasted_iota(jnp.int32, (C, C), 0)
    c16 = lax.broadcasted_iota(jnp.int32, (C, C), 1)
    striu16 = (r16 < c16).astype(jnp.float32)
    coff = lax.dot_general(csum, striu16, (((1,), (0,)), ((), ())),
                           preferred_element_type=jnp.float32)
    rB = lax.broadcasted_iota(jnp.int32, (C, T), 0)
    cB = lax.broadcasted_iota(jnp.int32, (C, T), 1)
    A = (cB // CH == rB).astype(jnp.float32)
    coff_hi = jnp.floor(coff * (1.0 / CH))
    coff_lo = coff - coff_hi * CH
    rank = within + (
        lax.dot_general(coff_hi, A, (((1,), (0,)), ((), ())),
                        preferred_element_type=jnp.float32) * CH
        + lax.dot_general(coff_lo, A, (((1,), (0,)), ((), ())),
                          preferred_element_type=jnp.float32))

    re8 = lax.broadcasted_iota(jnp.int32, (E, E), 0)
    ce8 = lax.broadcasted_iota(jnp.int32, (E, E), 1)
    stril = (ce8 < re8).astype(jnp.float32)
    stril_inc = (ce8 <= re8).astype(jnp.float32)
    jobs = jnp.floor((counts + (TB - 1)) * (1.0 / TB))
    jexc = lax.dot_general(stril, jobs, (((1,), (0,)), ((), ())),
                           preferred_element_type=jnp.float32)
    posf = jnp.sum(onehot * (rank + jexc * TB), axis=0, keepdims=True)
    pos_ref[...] = posf.astype(jnp.int32)

    jinc = lax.dot_general(stril_inc, jobs, (((1,), (0,)), ((), ())),
                           preferred_element_type=jnp.float32)
    total = jnp.sum(jobs, axis=0, keepdims=True)
    jv = lax.broadcasted_iota(jnp.int32, (E, NJOBS), 1).astype(jnp.float32)
    jcl = jnp.minimum(jv, total - 1.0)
    jexp = jnp.sum((jcl >= jinc).astype(jnp.float32), axis=0, keepdims=True)
    jtile = jcl[:1, :]
    validf = (jv[:1, :] < total).astype(jnp.float32)
    tbl_ref[...] = jnp.concatenate([jexp, jtile, validf],
                                   axis=0).astype(jnp.int32)


def _gating(flat_x, gate_w1, gate_b1, gate_w2):
    return pl.pallas_call(
        _gating_body,
        out_shape=[
            jax.ShapeDtypeStruct((1, T), jnp.int32),
            jax.ShapeDtypeStruct((3, NJOBS), jnp.int32),
            jax.ShapeDtypeStruct((1, 1), jnp.float32),
        ],
    )(flat_x, gate_w1, gate_b1.reshape(1, 2 * E), gate_w2)


def _expert_body(s_ref, x_ref, w_ref, b_ref, g_ref, bb_ref, pw_ref, pb_ref, o_ref):
    j = pl.program_id(0)

    @pl.when(s_ref[2, j] == 1)
    def _():
        xt = x_ref[...]
        w = w_ref[0]
        h = lax.dot_general(xt, w, (((1,), (1,)), ((), ())),
                            preferred_element_type=jnp.float32) + b_ref[0]
        glu = h[:, :K] * jax.nn.sigmoid(h[:, K:])
        m = jnp.mean(glu, axis=1, keepdims=True)
        v = jnp.mean((glu - m) * (glu - m), axis=1, keepdims=True)
        normed = (glu - m) * lax.rsqrt(v + LN_EPS)
        oe = normed * g_ref[0] + bb_ref[0]
        y = lax.dot_general(oe, pw_ref[...], (((1,), (1,)), ((), ())),
                            preferred_element_type=jnp.float32)
        o_ref[...] = y + pb_ref[...]


def _expert_call(tbl, x_sorted, ew1, eb1, ln_g, ln_b, proj_w, proj_b2):
    grid_spec = pltpu.PrefetchScalarGridSpec(
        num_scalar_prefetch=1,
        grid=(NJOBS,),
        in_specs=[
            pl.BlockSpec((TB, D), lambda j, s: (s[1, j], 0)),
            pl.BlockSpec((1, K2, D), lambda j, s: (s[0, j], 0, 0)),
            pl.BlockSpec((1, 1, K2), lambda j, s: (s[0, j], 0, 0)),
            pl.BlockSpec((1, 1, K), lambda j, s: (s[0, j], 0, 0)),
            pl.BlockSpec((1, 1, K), lambda j, s: (s[0, j], 0, 0)),
            pl.BlockSpec((D, K), lambda j, s: (0, 0)),
            pl.BlockSpec((1, D), lambda j, s: (0, 0)),
        ],
        out_specs=pl.BlockSpec((TB, D), lambda j, s: (s[1, j], 0)),
    )
    return pl.pallas_call(
        _expert_body,
        grid_spec=grid_spec,
        out_shape=jax.ShapeDtypeStruct((P, D), jnp.float32),
    )(tbl, x_sorted, ew1, eb1.reshape(E, 1, K2), ln_g.reshape(E, 1, K),
      ln_b.reshape(E, 1, K), proj_w, proj_b2)


def _sc_mesh():
    return plsc.VectorSubcoreMesh(core_axis_name="c", subcore_axis_name="s",
                                  num_cores=SC_CORES, num_subcores=SC_SUBCORES)


def _sc_dispatch(flat_x, pos):
    @functools.partial(
        pl.kernel,
        out_type=jax.ShapeDtypeStruct((P, D), jnp.float32),
        mesh=_sc_mesh(),
        scratch_types=[
            pltpu.VMEM((CHUNK,), jnp.int32),
            pltpu.VMEM((CHUNK, D), jnp.float32),
            pltpu.SemaphoreType.DMA,
        ],
    )
    def k(x_hbm, pos_hbm, out_hbm, idx_v, rows_v, sem):
        wid = lax.axis_index("s") * SC_CORES + lax.axis_index("c")
        base = wid * CHUNK
        pltpu.sync_copy(pos_hbm.at[pl.ds(base, CHUNK)], idx_v)
        pltpu.sync_copy(x_hbm.at[pl.ds(base, CHUNK)], rows_v)
        pltpu.async_copy(rows_v, out_hbm.at[idx_v], sem).wait()

    return k(flat_x, pos)


def _sc_unpermute(y_sorted, pos):
    @functools.partial(
        pl.kernel,
        out_type=jax.ShapeDtypeStruct((T, D), jnp.float32),
        mesh=_sc_mesh(),
        scratch_types=[
            pltpu.VMEM((CHUNK,), jnp.int32),
            pltpu.VMEM((CHUNK, D), jnp.float32),
            pltpu.SemaphoreType.DMA,
        ],
    )
    def k(y_hbm, pos_hbm, out_hbm, idx_v, rows_v, sem):
        wid = lax.axis_index("s") * SC_CORES + lax.axis_index("c")
        base = wid * CHUNK
        pltpu.sync_copy(pos_hbm.at[pl.ds(base, CHUNK)], idx_v)
        pltpu.async_copy(y_hbm.at[idx_v], rows_v, sem).wait()
        pltpu.sync_copy(rows_v, out_hbm.at[pl.ds(base, CHUNK)])

    return k(y_sorted, pos)


def kernel(x, gate_w1, gate_b1, gate_w2, ew1, eb1, ln_g, ln_b, proj_w, proj_b):
    bs, seq, d = x.shape
    flat_x = x.reshape(bs * seq, d)
    pos2, tbl, loss = _gating(flat_x, gate_w1, gate_b1, gate_w2)
    pos = pos2.reshape(T)
    x_sorted = _sc_dispatch(flat_x, pos)
    y_sorted = _expert_call(tbl, x_sorted, ew1, eb1, ln_g, ln_b,
                            proj_w, proj_b.reshape(1, D))
    y = _sc_unpermute(y_sorted, pos)
    return y.reshape(bs, seq, d), loss[0, 0]
